# speedup vs baseline: 5.6405x; 5.6405x over previous
import jax
import jax.numpy as jnp
from jax.experimental import pallas as pl
from jax.experimental.pallas import tpu as pltpu

O_N = 10000
NUM_OBJS_P1 = 180
NIMG = 8
EMB = 128
GDIM = 128
HID = 512
NOISE_DIM = 64

KPAD = 256
BLK = 1024
NPAD = 10240
NB = NPAD // BLK
OUTPAD = 128


def _prep_kernel(objs_ref, oti_ref, obj_emb_ref, gconv_W_ref, gconv_b_ref,
                 att_W_ref, W1a_ref, W1b_ref, b1_ref,
                 A_ref, Brep_ref, histT_ref):
    step = pl.program_id(0)

    @pl.when(step == 0)
    def _init():
        histT_ref[...] = jnp.zeros_like(histT_ref)

    objs_l = objs_ref[0]
    oti_l = oti_ref[0]
    ohT_obj = (jax.lax.broadcasted_iota(jnp.int32, (KPAD, BLK), 0)
               == objs_l).astype(jnp.float32)
    ohT_img = (jax.lax.broadcasted_iota(jnp.int32, (NIMG, BLK), 0)
               == oti_l).astype(jnp.float32)
    histT_ref[...] += jax.lax.dot_general(
        ohT_obj, ohT_img, (((1,), (1,)), ((), ())),
        preferred_element_type=jnp.float32)

    @pl.when(step == NB - 1)
    def _finish():
        histT = histT_ref[...]
        table_g = jnp.dot(obj_emb_ref[...], gconv_W_ref[...],
                          preferred_element_type=jnp.float32) + gconv_b_ref[...]
        table_a = jnp.dot(table_g, att_W_ref[...],
                          preferred_element_type=jnp.float32)
        counts = jax.lax.dot_general(
            histT, jnp.ones((KPAD, 1), jnp.float32),
            (((0,), (0,)), ((), ())), preferred_element_type=jnp.float32)
        counts = jnp.where(counts > 0.0, counts, 1.0)
        gc = jax.lax.dot_general(
            histT, table_a, (((0,), (0,)), ((), ())),
            preferred_element_type=jnp.float32) / counts
        tg = jnp.tanh(gc)
        sig = jax.nn.sigmoid(jax.lax.dot_general(
            table_g, tg, (((1,), (1,)), ((), ())),
            preferred_element_type=jnp.float32))
        w = histT * sig
        rep = jax.lax.dot_general(
            w, table_g, (((0,), (0,)), ((), ())),
            preferred_element_type=jnp.float32)
        A_ref[...] = jnp.dot(table_g, W1a_ref[...],
                             preferred_element_type=jnp.float32)
        Brep_ref[...] = jnp.dot(rep, W1b_ref[...],
                                preferred_element_type=jnp.float32) + b1_ref[...]


def _main_kernel(objs_ref, oti_ref, noise_ref, A_ref, Brep_ref, W1c_ref,
                 W2_ref, b2_ref, out_ref):
    objs_l = objs_ref[0]
    oti_l = oti_ref[0]
    ohT_obj = (jax.lax.broadcasted_iota(jnp.int32, (KPAD, BLK), 0)
               == objs_l).astype(jnp.float32)
    ohT_img = (jax.lax.broadcasted_iota(jnp.int32, (NIMG, BLK), 0)
               == oti_l).astype(jnp.float32)
    ha = jax.lax.dot_general(ohT_obj, A_ref[...], (((0,), (0,)), ((), ())),
                             preferred_element_type=jnp.float32)
    hb = jax.lax.dot_general(ohT_img, Brep_ref[...], (((0,), (0,)), ((), ())),
                             preferred_element_type=jnp.float32)
    hn = jnp.dot(noise_ref[...], W1c_ref[...],
                 preferred_element_type=jnp.float32)
    h = jax.nn.relu(ha + hb + hn)
    out_ref[...] = jnp.dot(h, W2_ref[...],
                           preferred_element_type=jnp.float32) + b2_ref[...]


@jax.jit
def _run(objs, noise, obj_to_img, obj_emb, gconv_W, gconv_b, att_W,
         box_W1, box_b1, box_W2, box_b2):
    objs_p = jnp.pad(objs.astype(jnp.int32), (0, NPAD - O_N),
                     constant_values=KPAD + 1).reshape(NB, 1, BLK)
    oti_p = jnp.pad(obj_to_img.astype(jnp.int32), (0, NPAD - O_N),
                    constant_values=NIMG + 1).reshape(NB, 1, BLK)
    noise_p = jnp.pad(noise, ((0, NPAD - O_N), (0, 0)))
    obj_emb_p = jnp.pad(obj_emb, ((0, KPAD - NUM_OBJS_P1), (0, 0)))
    W1a = box_W1[:GDIM]
    W1b = box_W1[GDIM:2 * GDIM]
    W1c = box_W1[2 * GDIM:]
    W2p = jnp.pad(box_W2, ((0, 0), (0, OUTPAD - 4)))
    b2p = jnp.pad(box_b2, (0, OUTPAD - 4)).reshape(1, OUTPAD)

    idx_spec = pl.BlockSpec((1, 1, BLK), lambda b: (b, 0, 0))

    def full(shape):
        return pl.BlockSpec(shape, lambda b, _s=shape: tuple(0 for _ in _s))

    A, Brep = pl.pallas_call(
        _prep_kernel,
        grid=(NB,),
        in_specs=[
            idx_spec, idx_spec,
            full((KPAD, EMB)), full((EMB, GDIM)), full((1, GDIM)),
            full((GDIM, GDIM)), full((GDIM, HID)), full((GDIM, HID)),
            full((1, HID)),
        ],
        out_specs=[full((KPAD, HID)), full((NIMG, HID))],
        out_shape=[jax.ShapeDtypeStruct((KPAD, HID), jnp.float32),
                   jax.ShapeDtypeStruct((NIMG, HID), jnp.float32)],
        scratch_shapes=[pltpu.VMEM((KPAD, NIMG), jnp.float32)],
    )(objs_p, oti_p, obj_emb_p, gconv_W, gconv_b.reshape(1, GDIM), att_W,
      W1a, W1b, box_b1.reshape(1, HID))

    out = pl.pallas_call(
        _main_kernel,
        grid=(NB,),
        in_specs=[
            idx_spec, idx_spec,
            pl.BlockSpec((BLK, NOISE_DIM), lambda b: (b, 0)),
            full((KPAD, HID)), full((NIMG, HID)), full((NOISE_DIM, HID)),
            full((HID, OUTPAD)), full((1, OUTPAD)),
        ],
        out_specs=pl.BlockSpec((BLK, OUTPAD), lambda b: (b, 0)),
        out_shape=jax.ShapeDtypeStruct((NPAD, OUTPAD), jnp.float32),
    )(objs_p, oti_p, noise_p, A, Brep, W1c, W2p, b2p)

    return out[:O_N, :4]


def kernel(objs, triples, noise, obj_to_img, obj_emb, pred_emb, gconv_W,
           gconv_b, att_W, box_W1, box_b1, box_W2, box_b2):
    del triples, pred_emb
    return _run(objs, noise, obj_to_img, obj_emb, gconv_W, gconv_b, att_W,
                box_W1, box_b1, box_W2, box_b2)


# speedup vs baseline: 6.5894x; 1.1682x over previous
import jax
import jax.numpy as jnp
from jax.experimental import pallas as pl
from jax.experimental.pallas import tpu as pltpu

O_N = 10000
NUM_OBJS_P1 = 180
NIMG = 8
EMB = 128
GDIM = 128
HID = 512
NOISE_DIM = 64

KPAD = 256
BLK = 1000
NB = O_N // BLK


def _prep_kernel(objs_ref, oti_ref, obj_emb_ref, gconv_W_ref, gconv_b_ref,
                 att_W_ref, W1a_ref, W1b_ref, b1_ref,
                 A_ref, Brep_ref, histT_ref):
    step = pl.program_id(0)

    @pl.when(step == 0)
    def _init():
        histT_ref[...] = jnp.zeros_like(histT_ref)

    objs_l = objs_ref[0]
    oti_l = oti_ref[0]
    ohT_obj = (jax.lax.broadcasted_iota(jnp.int32, (KPAD, BLK), 0)
               == objs_l).astype(jnp.float32)
    ohT_img = (jax.lax.broadcasted_iota(jnp.int32, (NIMG, BLK), 0)
               == oti_l).astype(jnp.float32)
    histT_ref[...] += jax.lax.dot_general(
        ohT_obj, ohT_img, (((1,), (1,)), ((), ())),
        preferred_element_type=jnp.float32)

    @pl.when(step == NB - 1)
    def _finish():
        histT = histT_ref[...]
        table_g = jnp.dot(obj_emb_ref[...], gconv_W_ref[...],
                          preferred_element_type=jnp.float32) + gconv_b_ref[...]
        table_a = jnp.dot(table_g, att_W_ref[...],
                          preferred_element_type=jnp.float32)
        counts = jax.lax.dot_general(
            histT, jnp.ones((KPAD, 1), jnp.float32),
            (((0,), (0,)), ((), ())), preferred_element_type=jnp.float32)
        counts = jnp.where(counts > 0.0, counts, 1.0)
        gc = jax.lax.dot_general(
            histT, table_a, (((0,), (0,)), ((), ())),
            preferred_element_type=jnp.float32) / counts
        tg = jnp.tanh(gc)
        sig = jax.nn.sigmoid(jax.lax.dot_general(
            table_g, tg, (((1,), (1,)), ((), ())),
            preferred_element_type=jnp.float32))
        w = histT * sig
        rep = jax.lax.dot_general(
            w, table_g, (((0,), (0,)), ((), ())),
            preferred_element_type=jnp.float32)
        A_ref[...] = jnp.dot(table_g, W1a_ref[...],
                             preferred_element_type=jnp.float32)
        Brep_ref[...] = jnp.dot(rep, W1b_ref[...],
                                preferred_element_type=jnp.float32) + b1_ref[...]


def _main_kernel(objs_ref, oti_ref, noise_ref, A_ref, Brep_ref, W1c_ref,
                 W2_ref, b2_ref, out_ref):
    objs_l = objs_ref[0]
    oti_l = oti_ref[0]
    ohT_obj = (jax.lax.broadcasted_iota(jnp.int32, (KPAD, BLK), 0)
               == objs_l).astype(jnp.float32)
    ohT_img = (jax.lax.broadcasted_iota(jnp.int32, (NIMG, BLK), 0)
               == oti_l).astype(jnp.float32)
    ha = jax.lax.dot_general(ohT_obj, A_ref[...], (((0,), (0,)), ((), ())),
                             preferred_element_type=jnp.float32)
    hb = jax.lax.dot_general(ohT_img, Brep_ref[...], (((0,), (0,)), ((), ())),
                             preferred_element_type=jnp.float32)
    hn = jnp.dot(noise_ref[...], W1c_ref[...],
                 preferred_element_type=jnp.float32)
    h = jax.nn.relu(ha + hb + hn)
    out_ref[...] = jnp.dot(h, W2_ref[...],
                           preferred_element_type=jnp.float32) + b2_ref[...]


@jax.jit
def _run(objs, noise, obj_to_img, obj_emb, gconv_W, gconv_b, att_W,
         box_W1, box_b1, box_W2, box_b2):
    objs_r = objs.astype(jnp.int32).reshape(NB, 1, BLK)
    oti_r = obj_to_img.astype(jnp.int32).reshape(NB, 1, BLK)
    obj_emb_p = jnp.pad(obj_emb, ((0, KPAD - NUM_OBJS_P1), (0, 0)))

    idx_spec = pl.BlockSpec((1, 1, BLK), lambda b: (b, 0, 0))

    def full(shape, idx=(0, 0)):
        return pl.BlockSpec(shape, lambda b, _i=idx: _i)

    A, Brep = pl.pallas_call(
        _prep_kernel,
        grid=(NB,),
        in_specs=[
            idx_spec, idx_spec,
            full((KPAD, EMB)), full((EMB, GDIM)), full((1, GDIM)),
            full((GDIM, GDIM)),
            full((GDIM, HID)),
            full((GDIM, HID), (1, 0)),
            full((1, HID)),
        ],
        out_specs=[full((KPAD, HID)), full((NIMG, HID))],
        out_shape=[jax.ShapeDtypeStruct((KPAD, HID), jnp.float32),
                   jax.ShapeDtypeStruct((NIMG, HID), jnp.float32)],
        scratch_shapes=[pltpu.VMEM((KPAD, NIMG), jnp.float32)],
    )(objs_r, oti_r, obj_emb_p, gconv_W, gconv_b.reshape(1, GDIM), att_W,
      box_W1, box_W1, box_b1.reshape(1, HID))

    out = pl.pallas_call(
        _main_kernel,
        grid=(NB,),
        in_specs=[
            idx_spec, idx_spec,
            pl.BlockSpec((BLK, NOISE_DIM), lambda b: (b, 0)),
            full((KPAD, HID)), full((NIMG, HID)),
            full((NOISE_DIM, HID), (4, 0)),
            full((HID, 4)), full((1, 4)),
        ],
        out_specs=pl.BlockSpec((BLK, 4), lambda b: (b, 0)),
        out_shape=jax.ShapeDtypeStruct((O_N, 4), jnp.float32),
    )(objs_r, oti_r, noise, A, Brep, box_W1, box_W2, box_b2.reshape(1, 4))

    return out


def kernel(objs, triples, noise, obj_to_img, obj_emb, pred_emb, gconv_W,
           gconv_b, att_W, box_W1, box_b1, box_W2, box_b2):
    del triples, pred_emb
    return _run(objs, noise, obj_to_img, obj_emb, gconv_W, gconv_b, att_W,
                box_W1, box_b1, box_W2, box_b2)


# speedup vs baseline: 7.4551x; 1.1314x over previous
import jax
import jax.numpy as jnp
from jax.experimental import pallas as pl
from jax.experimental.pallas import tpu as pltpu

O_N = 10000
NUM_OBJS_P1 = 180
NIMG = 8
EMB = 128
GDIM = 128
HID = 512
NOISE_DIM = 64

KPAD = 256
BLK = 2000
NB = O_N // BLK


def _fused_kernel(objs_ref, oti_ref, noise_ref, obj_emb_ref, gconv_W_ref,
                  gconv_b_ref, att_W_ref, W1a_ref, W1b_ref, W1c_ref, b1_ref,
                  W2_ref, b2_ref, out_ref,
                  histT_s, ohobj_s, ohimg_s, A_s, Brep_s):
    p = pl.program_id(0)
    b = pl.program_id(1)

    @pl.when(jnp.logical_and(p == 0, b == 0))
    def _init():
        histT_s[...] = jnp.zeros_like(histT_s)

    @pl.when(p == 0)
    def _phase0():
        objs_l = objs_ref[0]
        oti_l = oti_ref[0]
        ohT_obj = (jax.lax.broadcasted_iota(jnp.int32, (KPAD, BLK), 0)
                   == objs_l).astype(jnp.float32)
        ohT_img = (jax.lax.broadcasted_iota(jnp.int32, (NIMG, BLK), 0)
                   == oti_l).astype(jnp.float32)
        ohobj_s[b] = ohT_obj
        ohimg_s[b] = ohT_img
        histT_s[...] += jax.lax.dot_general(
            ohT_obj, ohT_img, (((1,), (1,)), ((), ())),
            preferred_element_type=jnp.float32)

        @pl.when(b == NB - 1)
        def _finish():
            histT = histT_s[...]
            table_g = jnp.dot(obj_emb_ref[...], gconv_W_ref[...],
                              preferred_element_type=jnp.float32) + gconv_b_ref[...]
            table_a = jnp.dot(table_g, att_W_ref[...],
                              preferred_element_type=jnp.float32)
            counts = jax.lax.dot_general(
                histT, jnp.ones((KPAD, 1), jnp.float32),
                (((0,), (0,)), ((), ())), preferred_element_type=jnp.float32)
            counts = jnp.where(counts > 0.0, counts, 1.0)
            gc = jax.lax.dot_general(
                histT, table_a, (((0,), (0,)), ((), ())),
                preferred_element_type=jnp.float32) / counts
            tg = jnp.tanh(gc)
            sig = jax.nn.sigmoid(jax.lax.dot_general(
                table_g, tg, (((1,), (1,)), ((), ())),
                preferred_element_type=jnp.float32))
            w = histT * sig
            rep = jax.lax.dot_general(
                w, table_g, (((0,), (0,)), ((), ())),
                preferred_element_type=jnp.float32)
            A_s[...] = jnp.dot(table_g, W1a_ref[...],
                               preferred_element_type=jnp.float32)
            Brep_s[...] = jnp.dot(rep, W1b_ref[...],
                                  preferred_element_type=jnp.float32) + b1_ref[...]

    @pl.when(p == 1)
    def _phase1():
        ha = jax.lax.dot_general(ohobj_s[b], A_s[...],
                                 (((0,), (0,)), ((), ())),
                                 preferred_element_type=jnp.float32)
        hb = jax.lax.dot_general(ohimg_s[b], Brep_s[...],
                                 (((0,), (0,)), ((), ())),
                                 preferred_element_type=jnp.float32)
        hn = jnp.dot(noise_ref[...], W1c_ref[...],
                     preferred_element_type=jnp.float32)
        h = jax.nn.relu(ha + hb + hn)
        out_ref[...] = jnp.dot(h, W2_ref[...],
                               preferred_element_type=jnp.float32) + b2_ref[...]


@jax.jit
def _run(objs, noise, obj_to_img, obj_emb, gconv_W, gconv_b, att_W,
         box_W1, box_b1, box_W2, box_b2):
    objs_r = objs.astype(jnp.int32).reshape(NB, 1, BLK)
    oti_r = obj_to_img.astype(jnp.int32).reshape(NB, 1, BLK)
    obj_emb_p = jnp.pad(obj_emb, ((0, KPAD - NUM_OBJS_P1), (0, 0)))

    idx_spec = pl.BlockSpec((1, 1, BLK), lambda p, b: (b, 0, 0))

    def full(shape, idx=(0, 0)):
        return pl.BlockSpec(shape, lambda p, b, _i=idx: _i)

    out = pl.pallas_call(
        _fused_kernel,
        grid=(2, NB),
        in_specs=[
            idx_spec, idx_spec,
            pl.BlockSpec((BLK, NOISE_DIM), lambda p, b: (b * p, 0)),
            full((KPAD, EMB)), full((EMB, GDIM)), full((1, GDIM)),
            full((GDIM, GDIM)),
            full((GDIM, HID)),
            full((GDIM, HID), (1, 0)),
            full((NOISE_DIM, HID), (4, 0)),
            full((1, HID)),
            full((HID, 4)), full((1, 4)),
        ],
        out_specs=pl.BlockSpec((BLK, 4), lambda p, b: (b, 0)),
        out_shape=jax.ShapeDtypeStruct((O_N, 4), jnp.float32),
        scratch_shapes=[
            pltpu.VMEM((KPAD, NIMG), jnp.float32),
            pltpu.VMEM((NB, KPAD, BLK), jnp.float32),
            pltpu.VMEM((NB, NIMG, BLK), jnp.float32),
            pltpu.VMEM((KPAD, HID), jnp.float32),
            pltpu.VMEM((NIMG, HID), jnp.float32),
        ],
    )(objs_r, oti_r, noise, obj_emb_p, gconv_W, gconv_b.reshape(1, GDIM),
      att_W, box_W1, box_W1, box_W1, box_b1.reshape(1, HID), box_W2,
      box_b2.reshape(1, 4))

    return out


def kernel(objs, triples, noise, obj_to_img, obj_emb, pred_emb, gconv_W,
           gconv_b, att_W, box_W1, box_b1, box_W2, box_b2):
    del triples, pred_emb
    return _run(objs, noise, obj_to_img, obj_emb, gconv_W, gconv_b, att_W,
                box_W1, box_b1, box_W2, box_b2)
